# baseline (speedup 1.0000x reference)
import jax
import jax.numpy as jnp
from jax.experimental import pallas as pl
from jax.experimental.pallas import tpu as pltpu

_ALPHA = 0.5
_BB = 8


def _align_body(x_ref, p_ref, p2_ref, o_ref, xkeep, feat2):
    i = pl.program_id(0)
    nb = pl.num_programs(0) - 1
    cur = jax.lax.rem(i, 2)
    prv = 1 - cur

    @pl.when(i < nb)
    def _pool():
        xb = x_ref[...]
        xkeep[cur] = xb
        feat2[cur] = jnp.sum(xb, axis=1) * (1.0 / xb.shape[1])

    @pl.when(i > 0)
    def _align():
        feat = feat2[prv]
        f2 = jnp.sum(feat * feat, axis=1, keepdims=True)
        dots = jax.lax.dot_general(
            feat.astype(jnp.bfloat16), p_ref[...], (((1,), (1,)), ((), ())),
            preferred_element_type=jnp.float32)
        d2 = jnp.maximum((f2 + p2_ref[...]) - 2.0 * dots, 0.0)
        m = jnp.min(d2, axis=1, keepdims=True)
        ii = jax.lax.broadcasted_iota(jnp.int32, d2.shape, 1)
        idx = jnp.min(jnp.where(d2 <= m, ii, jnp.int32(d2.shape[1])),
                      axis=1, keepdims=True)
        onehot = (ii == idx).astype(jnp.bfloat16)
        nearest = jax.lax.dot_general(
            onehot, p_ref[...], (((1,), (0,)), ((), ())),
            preferred_element_type=jnp.float32)
        delta = _ALPHA * (nearest - feat)
        o_ref[...] = xkeep[prv] + delta[:, None, :]


def kernel(x, prototypes):
    B, C, H, W = x.shape
    K = prototypes.shape[0]
    HW = H * W
    NB = B // _BB
    xt = x.transpose(0, 2, 3, 1).reshape(B, HW, C)
    p_bf = prototypes.astype(jnp.bfloat16)
    p2 = jnp.sum(prototypes * prototypes, axis=1)[None, :]
    out_t = pl.pallas_call(
        _align_body,
        grid=(NB + 1,),
        in_specs=[
            pl.BlockSpec((_BB, HW, C), lambda i: (jnp.minimum(i, NB - 1), 0, 0)),
            pl.BlockSpec((K, C), lambda i: (0, 0)),
            pl.BlockSpec((1, K), lambda i: (0, 0)),
        ],
        out_specs=pl.BlockSpec((_BB, HW, C),
                               lambda i: (jnp.maximum(i - 1, 0), 0, 0)),
        out_shape=jax.ShapeDtypeStruct((B, HW, C), x.dtype),
        scratch_shapes=[
            pltpu.VMEM((2, _BB, HW, C), jnp.float32),
            pltpu.VMEM((2, _BB, C), jnp.float32),
        ],
        compiler_params=pltpu.CompilerParams(
            dimension_semantics=("arbitrary",)),
    )(xt, p_bf, p2)
    return out_t.reshape(B, H, W, C).transpose(0, 3, 1, 2)
